# baseline (speedup 1.0000x reference)
import functools

import jax
import jax.numpy as jnp
from jax import lax
from jax.experimental import pallas as pl
from jax.experimental.pallas import tpu as pltpu
from jax.experimental.pallas import tpu_sc as plsc

N_ROWS = 16384
N_COLS = 200
TOTAL = N_ROWS * N_COLS
TABLE = 1_000_000
NC = 2
NS = 16
NW = NC * NS
PER_W = TOTAL // NW
CHUNK = 12_800
NCHUNK = PER_W // CHUNK
T_SLICE = 65536
TAIL = TABLE - 15 * T_SLICE
PIECE = 16384


def _sigmoid_table(alpha):
    def sig_kernel(a_ref, o_ref):
        o_ref[...] = jax.nn.sigmoid(a_ref[...])

    return pl.pallas_call(
        sig_kernel,
        out_shape=jax.ShapeDtypeStruct((TABLE,), jnp.float32),
    )(alpha)


def _gather(sig, idx_flat):
    mesh = plsc.VectorSubcoreMesh(core_axis_name="c", subcore_axis_name="s")

    @functools.partial(
        pl.kernel,
        mesh=mesh,
        out_type=jax.ShapeDtypeStruct((TOTAL,), jnp.float32),
        scratch_types=[
            pltpu.VMEM_SHARED((TABLE,), jnp.float32),
            pltpu.VMEM((PIECE,), jnp.float32),
            pltpu.VMEM((CHUNK,), jnp.int32),
            pltpu.VMEM((CHUNK,), jnp.int32),
            pltpu.VMEM((CHUNK,), jnp.float32),
            pltpu.VMEM((CHUNK,), jnp.float32),
            pltpu.SemaphoreType.DMA,
            pltpu.SemaphoreType.DMA,
            pltpu.SemaphoreType.DMA,
            pltpu.SemaphoreType.DMA,
            pltpu.SemaphoreType.DMA,
        ],
    )
    def k(sig_hbm, idx_hbm, out_hbm,
          table_s, stage_v, idx_0, idx_1, val_0, val_1,
          si0, si1, sg, so0, so1):
        idx_v = [idx_0, idx_1]
        val_v = [val_0, val_1]
        s_i = [si0, si1]
        s_o = [so0, so1]

        cid = lax.axis_index("c")
        tid = lax.axis_index("s")
        wid = tid * NC + cid
        base = wid * PER_W

        def idx_copy(c):
            return pltpu.async_copy(
                idx_hbm.at[pl.ds(base + c * CHUNK, CHUNK)],
                idx_v[c % 2], s_i[c % 2])

        h_i = {}
        h_i[0] = idx_copy(0)
        h_i[1] = idx_copy(1)

        def stage_piece(off, length):
            pltpu.sync_copy(sig_hbm.at[pl.ds(off, length)],
                            stage_v.at[pl.ds(0, length)])
            pltpu.sync_copy(stage_v.at[pl.ds(0, length)],
                            table_s.at[pl.ds(off, length)])

        @pl.when(tid < NS - 1)
        def _():
            for p in range(T_SLICE // PIECE):
                stage_piece(tid * T_SLICE + p * PIECE, PIECE)

        @pl.when(tid == NS - 1)
        def _():
            stage_piece((NS - 1) * T_SLICE, PIECE)
            stage_piece((NS - 1) * T_SLICE + PIECE, TAIL - PIECE)

        plsc.subcore_barrier()

        def gather(c):
            return pltpu.async_copy(
                table_s.at[idx_v[c % 2]], val_v[c % 2], sg)

        def out_copy(c):
            return pltpu.async_copy(
                val_v[c % 2],
                out_hbm.at[pl.ds(base + c * CHUNK, CHUNK)], s_o[c % 2])

        h_o = {}
        for c in range(NCHUNK):
            if c >= 2:
                h_o[c - 2].wait()
            h_i[c].wait()
            gather(c).wait()
            h_o[c] = out_copy(c)
            if c + 2 < NCHUNK:
                h_i[c + 2] = idx_copy(c + 2)
        h_o[NCHUNK - 2].wait()
        h_o[NCHUNK - 1].wait()

    return k(sig, idx_flat)


def kernel(idx, alpha):
    idx_flat = jnp.minimum(idx.reshape(-1), TABLE - 1)
    out_flat = _gather(_sigmoid_table(alpha), idx_flat)
    return jnp.maximum(out_flat.reshape(N_ROWS, N_COLS), 0.0)


# speedup vs baseline: 1.1354x; 1.1354x over previous
import functools

import jax
import jax.numpy as jnp
from jax import lax
from jax.experimental import pallas as pl
from jax.experimental.pallas import tpu as pltpu
from jax.experimental.pallas import tpu_sc as plsc

N_ROWS = 16384
N_COLS = 200
TOTAL = N_ROWS * N_COLS
TABLE = 1_000_000
NC = 2
NS = 16
NW = NC * NS
PER_W = TOTAL // NW
CHUNK = 12_800
NCHUNK = PER_W // CHUNK
T_SLICE = 65536
TAIL = TABLE - 15 * T_SLICE
PIECE = 16384


def _sigmoid_table(alpha):
    def sig_kernel(a_ref, o_ref):
        o_ref[...] = jax.nn.sigmoid(a_ref[...])

    return pl.pallas_call(
        sig_kernel,
        out_shape=jax.ShapeDtypeStruct((TABLE,), jnp.float32),
    )(alpha)


def _gather(sig, idx_flat):
    mesh = plsc.VectorSubcoreMesh(core_axis_name="c", subcore_axis_name="s")

    @functools.partial(
        pl.kernel,
        mesh=mesh,
        out_type=jax.ShapeDtypeStruct((TOTAL,), jnp.float32),
        scratch_types=[
            pltpu.VMEM_SHARED((TABLE,), jnp.float32),
            pltpu.VMEM((PIECE,), jnp.float32),
            pltpu.VMEM((CHUNK,), jnp.int32),
            pltpu.VMEM((CHUNK,), jnp.int32),
            pltpu.VMEM((CHUNK,), jnp.float32),
            pltpu.VMEM((CHUNK,), jnp.float32),
            pltpu.SemaphoreType.DMA,
            pltpu.SemaphoreType.DMA,
            pltpu.SemaphoreType.DMA,
            pltpu.SemaphoreType.DMA,
            pltpu.SemaphoreType.DMA,
            pltpu.SemaphoreType.DMA,
        ],
    )
    def k(sig_hbm, idx_hbm, out_hbm,
          table_s, stage_v, idx_0, idx_1, val_0, val_1,
          si0, si1, sg0, sg1, so0, so1):
        idx_v = [idx_0, idx_1]
        val_v = [val_0, val_1]
        s_i = [si0, si1]
        s_o = [so0, so1]

        cid = lax.axis_index("c")
        tid = lax.axis_index("s")
        wid = tid * NC + cid
        base = wid * PER_W

        def idx_copy(c):
            return pltpu.async_copy(
                idx_hbm.at[pl.ds(base + c * CHUNK, CHUNK)],
                idx_v[c % 2], s_i[c % 2])

        h_i = {}
        h_i[0] = idx_copy(0)
        h_i[1] = idx_copy(1)

        def stage_piece(off, length):
            pltpu.sync_copy(sig_hbm.at[pl.ds(off, length)],
                            stage_v.at[pl.ds(0, length)])
            pltpu.sync_copy(stage_v.at[pl.ds(0, length)],
                            table_s.at[pl.ds(off, length)])

        @pl.when(tid < NS - 1)
        def _():
            for p in range(T_SLICE // PIECE):
                stage_piece(tid * T_SLICE + p * PIECE, PIECE)

        @pl.when(tid == NS - 1)
        def _():
            stage_piece((NS - 1) * T_SLICE, PIECE)
            stage_piece((NS - 1) * T_SLICE + PIECE, TAIL - PIECE)

        plsc.subcore_barrier()

        HALF = CHUNK // 2

        def gather(c):
            b = c % 2
            h0 = pltpu.async_copy(
                table_s.at[idx_v[b].at[pl.ds(0, HALF)]],
                val_v[b].at[pl.ds(0, HALF)], sg0)
            h1 = pltpu.async_copy(
                table_s.at[idx_v[b].at[pl.ds(HALF, HALF)]],
                val_v[b].at[pl.ds(HALF, HALF)], sg1)
            return (h0, h1)

        def out_copy(c):
            return pltpu.async_copy(
                val_v[c % 2],
                out_hbm.at[pl.ds(base + c * CHUNK, CHUNK)], s_o[c % 2])

        h_o = {}
        for c in range(NCHUNK):
            if c >= 2:
                h_o[c - 2].wait()
            h_i[c].wait()
            g0, g1 = gather(c)
            g0.wait()
            g1.wait()
            h_o[c] = out_copy(c)
            if c + 2 < NCHUNK:
                h_i[c + 2] = idx_copy(c + 2)
        h_o[NCHUNK - 2].wait()
        h_o[NCHUNK - 1].wait()

    return k(sig, idx_flat)


def kernel(idx, alpha):
    out_flat = _gather(_sigmoid_table(alpha), idx.reshape(-1))
    return out_flat.reshape(N_ROWS, N_COLS)


# speedup vs baseline: 1.1699x; 1.0303x over previous
import functools

import jax
import jax.numpy as jnp
from jax import lax
from jax.experimental import pallas as pl
from jax.experimental.pallas import tpu as pltpu
from jax.experimental.pallas import tpu_sc as plsc

N_ROWS = 16384
N_COLS = 200
TOTAL = N_ROWS * N_COLS
TABLE = 1_000_000
NC = 2
NS = 16
NW = NC * NS
PER_W = TOTAL // NW
CHUNK = 12_800
NCHUNK = PER_W // CHUNK
T_SLICE = 65536
TAIL = TABLE - 15 * T_SLICE
PIECE = 16384


def _sigmoid_table(alpha):
    def sig_kernel(a_ref, o_ref):
        o_ref[...] = jax.nn.sigmoid(a_ref[...])

    return pl.pallas_call(
        sig_kernel,
        out_shape=jax.ShapeDtypeStruct((TABLE,), jnp.float32),
    )(alpha)


def _gather(sig, idx_flat):
    mesh = plsc.VectorSubcoreMesh(core_axis_name="c", subcore_axis_name="s")

    @functools.partial(
        pl.kernel,
        mesh=mesh,
        out_type=jax.ShapeDtypeStruct((TOTAL,), jnp.float32),
        scratch_types=[
            pltpu.VMEM_SHARED((TABLE,), jnp.float32),
            pltpu.VMEM((1024,), jnp.float32),
            pltpu.VMEM((CHUNK,), jnp.int32),
            pltpu.VMEM((CHUNK,), jnp.int32),
            pltpu.VMEM((CHUNK,), jnp.float32),
            pltpu.VMEM((CHUNK,), jnp.float32),
            pltpu.SemaphoreType.DMA,
            pltpu.SemaphoreType.DMA,
            pltpu.SemaphoreType.DMA,
            pltpu.SemaphoreType.DMA,
            pltpu.SemaphoreType.DMA,
            pltpu.SemaphoreType.DMA,
        ],
    )
    def k(sig_hbm, idx_hbm, out_hbm,
          table_s, stage_v, idx_0, idx_1, val_0, val_1,
          si0, si1, sg0, sg1, so0, so1):
        idx_v = [idx_0, idx_1]
        val_v = [val_0, val_1]
        s_i = [si0, si1]
        s_o = [so0, so1]

        cid = lax.axis_index("c")
        tid = lax.axis_index("s")
        wid = tid * NC + cid
        base = wid * PER_W

        def idx_copy(c):
            return pltpu.async_copy(
                idx_hbm.at[pl.ds(base + c * CHUNK, CHUNK)],
                idx_v[c % 2], s_i[c % 2])

        h_i = {}
        h_i[0] = idx_copy(0)
        h_i[1] = idx_copy(1)

        @pl.when(tid < NS - 1)
        def _():
            pltpu.sync_copy(sig_hbm.at[pl.ds(tid * T_SLICE, T_SLICE)],
                            table_s.at[pl.ds(tid * T_SLICE, T_SLICE)])

        @pl.when(tid == NS - 1)
        def _():
            t0 = (NS - 1) * T_SLICE
            pltpu.sync_copy(sig_hbm.at[pl.ds(t0, PIECE)],
                            table_s.at[pl.ds(t0, PIECE)])
            rem = TAIL - PIECE
            pltpu.sync_copy(sig_hbm.at[pl.ds(t0 + PIECE, rem)],
                            stage_v.at[pl.ds(0, rem)])
            pltpu.sync_copy(stage_v.at[pl.ds(0, rem)],
                            table_s.at[pl.ds(t0 + PIECE, rem)])

        plsc.subcore_barrier()

        HALF = CHUNK // 2

        def gather(c):
            b = c % 2
            h0 = pltpu.async_copy(
                table_s.at[idx_v[b].at[pl.ds(0, HALF)]],
                val_v[b].at[pl.ds(0, HALF)], sg0)
            h1 = pltpu.async_copy(
                table_s.at[idx_v[b].at[pl.ds(HALF, HALF)]],
                val_v[b].at[pl.ds(HALF, HALF)], sg1)
            return (h0, h1)

        def out_copy(c):
            return pltpu.async_copy(
                val_v[c % 2],
                out_hbm.at[pl.ds(base + c * CHUNK, CHUNK)], s_o[c % 2])

        h_o = {}
        for c in range(NCHUNK):
            if c >= 2:
                h_o[c - 2].wait()
            h_i[c].wait()
            g0, g1 = gather(c)
            g0.wait()
            g1.wait()
            h_o[c] = out_copy(c)
            if c + 2 < NCHUNK:
                h_i[c + 2] = idx_copy(c + 2)
        h_o[NCHUNK - 2].wait()
        h_o[NCHUNK - 1].wait()

    return k(sig, idx_flat)


def kernel(idx, alpha):
    out_flat = _gather(_sigmoid_table(alpha), idx.reshape(-1))
    return out_flat.reshape(N_ROWS, N_COLS)
